# Initial kernel scaffold; baseline (speedup 1.0000x reference)
import jax
import jax.numpy as jnp
from jax.experimental import pallas as pl


def kernel(symbols, embedding):
    raise NotImplementedError("write your pallas kernel here")


# speedup vs baseline: 26.6669x; 26.6669x over previous
import functools

import jax
import jax.numpy as jnp
from jax import lax
from jax.experimental import pallas as pl
from jax.experimental.pallas import tpu as pltpu
from jax.experimental.pallas import tpu_sc as plsc

_NC = 2
_NS = 16
_NW = _NC * _NS
_LANES = 16

_CHUNK = 6400


def _sc_lookup(sym_flat, tre, tim):
    n = sym_flat.shape[0]
    n_per_w = n // _NW
    n_chunks = n_per_w // _CHUNK
    assert n_per_w * _NW == n and n_chunks * _CHUNK == n_per_w

    mesh = plsc.VectorSubcoreMesh(core_axis_name="c", subcore_axis_name="s")

    @functools.partial(
        pl.kernel,
        out_type=(
            jax.ShapeDtypeStruct((n,), jnp.float32),
            jax.ShapeDtypeStruct((n,), jnp.float32),
        ),
        mesh=mesh,
        compiler_params=pltpu.CompilerParams(needs_layout_passes=False),
        scratch_types=[
            pltpu.VMEM((_CHUNK,), jnp.int32),
            pltpu.VMEM((_LANES,), jnp.float32),
            pltpu.VMEM((_LANES,), jnp.float32),
            pltpu.VMEM((_CHUNK,), jnp.float32),
            pltpu.VMEM((_CHUNK,), jnp.float32),
        ],
    )
    def k(sym_hbm, tre_hbm, tim_hbm, re_hbm, im_hbm,
          idx_v, tre_v, tim_v, re_v, im_v):
        wid = lax.axis_index("s") * _NC + lax.axis_index("c")
        base = wid * n_per_w
        pltpu.sync_copy(tre_hbm, tre_v)
        pltpu.sync_copy(tim_hbm, tim_v)

        def chunk_body(c, carry):
            off = base + c * _CHUNK
            pltpu.sync_copy(sym_hbm.at[pl.ds(off, _CHUNK)], idx_v)

            def inner(i, carry2):
                idx = idx_v[pl.ds(i * _LANES, _LANES)]
                re_v[pl.ds(i * _LANES, _LANES)] = plsc.load_gather(tre_v, [idx])
                im_v[pl.ds(i * _LANES, _LANES)] = plsc.load_gather(tim_v, [idx])
                return carry2

            lax.fori_loop(0, _CHUNK // _LANES, inner, 0)
            pltpu.sync_copy(re_v, re_hbm.at[pl.ds(off, _CHUNK)])
            pltpu.sync_copy(im_v, im_hbm.at[pl.ds(off, _CHUNK)])
            return carry

        lax.fori_loop(0, n_chunks, chunk_body, 0)

    return k(sym_flat, tre, tim)


def kernel(symbols, embedding):
    b, l = symbols.shape
    n = b * l
    sym = symbols.reshape(n).astype(jnp.int32)
    re, im = _sc_lookup(sym, embedding[:, 0], embedding[:, 1])
    return lax.complex(re.reshape(b, l), im.reshape(b, l))


# speedup vs baseline: 36.6870x; 1.3758x over previous
import functools

import jax
import jax.numpy as jnp
from jax import lax
from jax.experimental import pallas as pl
from jax.experimental.pallas import tpu as pltpu
from jax.experimental.pallas import tpu_sc as plsc

_NC = 2
_NS = 16
_NW = _NC * _NS
_LANES = 16

_CHUNK = 6400


def _sc_lookup(sym_flat, tre, tim):
    n = sym_flat.shape[0]
    n_per_w = n // _NW
    n_chunks = n_per_w // _CHUNK
    assert n_per_w * _NW == n and n_chunks * _CHUNK == n_per_w

    mesh = plsc.VectorSubcoreMesh(core_axis_name="c", subcore_axis_name="s")

    @functools.partial(
        pl.kernel,
        out_type=(
            jax.ShapeDtypeStruct((n,), jnp.float32),
            jax.ShapeDtypeStruct((n,), jnp.float32),
        ),
        mesh=mesh,
        compiler_params=pltpu.CompilerParams(needs_layout_passes=False),
        scratch_types=[
            pltpu.VMEM((_CHUNK,), jnp.int32),
            pltpu.VMEM((_LANES,), jnp.float32),
            pltpu.VMEM((_LANES,), jnp.float32),
            pltpu.VMEM((_CHUNK,), jnp.float32),
            pltpu.VMEM((_CHUNK,), jnp.float32),
        ],
    )
    def k(sym_hbm, tre_hbm, tim_hbm, re_hbm, im_hbm,
          idx_v, tre_v, tim_v, re_v, im_v):
        wid = lax.axis_index("s") * _NC + lax.axis_index("c")
        base = wid * n_per_w
        pltpu.sync_copy(tre_hbm, tre_v)
        pltpu.sync_copy(tim_hbm, tim_v)

        def chunk_body(c, carry):
            off = base + c * _CHUNK
            pltpu.sync_copy(sym_hbm.at[pl.ds(off, _CHUNK)], idx_v)

            def inner(i, carry2):
                idx = idx_v[pl.ds(i * _LANES, _LANES)]
                re_v[pl.ds(i * _LANES, _LANES)] = plsc.load_gather(tre_v, [idx])
                im_v[pl.ds(i * _LANES, _LANES)] = plsc.load_gather(tim_v, [idx])
                return carry2

            lax.fori_loop(0, _CHUNK // _LANES, inner, 0)
            pltpu.sync_copy(re_v, re_hbm.at[pl.ds(off, _CHUNK)])
            pltpu.sync_copy(im_v, im_hbm.at[pl.ds(off, _CHUNK)])
            return carry

        lax.fori_loop(0, n_chunks, chunk_body, 0)

    return k(sym_flat, tre, tim)


def kernel(symbols, embedding):
    b, l = symbols.shape
    n = b * l
    sym = symbols.T.reshape(n).astype(jnp.int32)
    re, im = _sc_lookup(sym, embedding[:, 0], embedding[:, 1])
    return lax.complex(re, im).reshape(l, b).T
